# baseline (speedup 1.0000x reference)
import jax
import jax.numpy as jnp
from jax import lax
from jax.experimental import pallas as pl
from jax.experimental.pallas import tpu as pltpu

F = 768
N = 8192
VF = 64
TILE = 512
NT = N // TILE
ACT = 2 * N + 2


def _body(x_ref, afW1_ref, afb1_ref, afW2r_ref, afb2_ref,
          asW1b_ref, asb1_ref, row0_ref, asW2r_ref, asb2_ref,
          at1W_ref, at2W_ref, at2b_ref, vW_ref, vb_ref,
          g1_ref, g2_ref, g3_ref,
          critic_ref, actor_ref,
          base2_scr, lg1_scr, lg2_scr, pooled_scr, af_scr):
    i = pl.program_id(0)

    @pl.when(i < NT)
    def _phase_a():
        x = x_ref[...]
        h1 = jnp.maximum(
            jnp.dot(x, afW1_ref[...], preferred_element_type=jnp.float32)
            + afb1_ref[...], 0.0)
        lg1_scr[:, pl.ds(i * TILE, TILE)] = lax.dot_general(
            afW2r_ref[...], h1.astype(jnp.bfloat16),
            (((1,), (1,)), ((), ())), preferred_element_type=jnp.float32)

        base2_scr[i] = jnp.dot(
            x, asW1b_ref[...], preferred_element_type=jnp.float32) \
            + asb1_ref[...]

        ht = jnp.maximum(
            jnp.dot(x, at1W_ref[...], preferred_element_type=jnp.float32),
            0.0)
        psum = jnp.sum(ht, axis=0, keepdims=True)

        @pl.when(i == 0)
        def _():
            pooled_scr[...] = jnp.zeros_like(pooled_scr)

        pooled_scr[...] += psum

        critic_ref[...] = jnp.maximum(
            jnp.dot(x, vW_ref[...], preferred_element_type=jnp.float32)
            + vb_ref[...], 0.0)

        @pl.when(i == NT - 1)
        def _():
            z = lg1_scr[...] + afb2_ref[0, 0] + g1_ref[...]
            m = jnp.max(z)
            idx = lax.broadcasted_iota(jnp.int32, (1, N), 1)
            af_scr[0, 0] = jnp.min(jnp.where(z == m, idx, N))

    @pl.when(i >= NT)
    def _phase_b():
        j = i - NT
        a_f = af_scr[0, 0]
        c = a_f.astype(jnp.float32)
        h2 = jnp.maximum(base2_scr[j] + c * row0_ref[...], 0.0)
        lg2_scr[:, pl.ds(j * TILE, TILE)] = lax.dot_general(
            asW2r_ref[...], h2.astype(jnp.bfloat16),
            (((1,), (1,)), ((), ())), preferred_element_type=jnp.float32)

        @pl.when(i == 2 * NT - 1)
        def _():
            idx = lax.broadcasted_iota(jnp.int32, (1, N), 1)
            z2 = jnp.where(idx == a_f, -jnp.inf,
                           lg2_scr[...] + asb2_ref[0, 0] + g2_ref[...])
            m2 = jnp.max(z2)
            a_s = jnp.min(jnp.where(z2 == m2, idx, N))

            lgt = jnp.dot(pooled_scr[...], at2W_ref[...],
                          preferred_element_type=jnp.float32) + at2b_ref[...]
            zt = lgt + g3_ref[...]
            a_t = jnp.where(zt[0, 1] > zt[0, 0], 1, 0)

            f1 = jnp.where(a_f == 0, 1.0, 0.0).astype(jnp.float32)
            f2 = jnp.where(a_s == 0, 1.0, 0.0).astype(jnp.float32)
            f3 = jnp.where(a_t == 0, 1.0, 0.0).astype(jnp.float32)
            aidx = lax.broadcasted_iota(jnp.int32, (1, ACT), 1)
            actor_ref[...] = jnp.where(
                aidx < N, f1, jnp.where(aidx < 2 * N, f2, f3))


def _full(shape):
    return pl.BlockSpec(shape, lambda i: tuple(0 for _ in shape))


def _smem11():
    return pl.BlockSpec((1, 1), lambda i: (0, 0), memory_space=pltpu.SMEM)


def kernel(features, af_W1, af_b1, af_W2, af_b2, as_W1, as_b1, as_W2,
           as_b2, at1_W, at2_W, at2_b, v_W, v_b):
    key = jax.random.key(42)
    k1, k2, k3 = jax.random.split(key, 3)
    g1 = jax.random.gumbel(k1, (1, N), jnp.float32)
    g2 = jax.random.gumbel(k2, (1, N), jnp.float32)
    g3 = jax.random.gumbel(k3, (1, 2), jnp.float32)

    bf = jnp.bfloat16
    x = features.reshape(N, F).astype(bf)
    afW1 = af_W1.astype(bf)
    afb1 = af_b1.reshape(1, F)
    afW2r = af_W2.reshape(1, F).astype(bf)
    afb2 = af_b2.reshape(1, 1)
    asW1b = as_W1[1:].astype(bf)
    row0 = as_W1[0].reshape(1, F)
    asb1 = as_b1.reshape(1, F)
    asW2r = as_W2.reshape(1, F).astype(bf)
    asb2 = as_b2.reshape(1, 1)
    at1W = at1_W.astype(bf)
    at2b = at2_b.reshape(1, 2)
    vW = v_W.astype(bf)
    vb = v_b.reshape(1, VF)

    critic, actor = pl.pallas_call(
        _body,
        grid=(2 * NT,),
        in_specs=[
            pl.BlockSpec((TILE, F), lambda i: (jnp.minimum(i, NT - 1), 0)),
            _full((F, F)), _full((1, F)), _full((1, F)), _smem11(),
            _full((F, F)), _full((1, F)), _full((1, F)), _full((1, F)),
            _smem11(),
            _full((F, F)), _full((F, 2)), _full((1, 2)),
            _full((F, VF)), _full((1, VF)),
            _full((1, N)), _full((1, N)), _full((1, 2)),
        ],
        out_specs=[
            pl.BlockSpec((TILE, VF), lambda i: (jnp.minimum(i, NT - 1), 0)),
            _full((1, ACT)),
        ],
        out_shape=[
            jax.ShapeDtypeStruct((N, VF), jnp.float32),
            jax.ShapeDtypeStruct((1, ACT), jnp.float32),
        ],
        scratch_shapes=[
            pltpu.VMEM((NT, TILE, F), jnp.float32),
            pltpu.VMEM((1, N), jnp.float32),
            pltpu.VMEM((1, N), jnp.float32),
            pltpu.VMEM((1, F), jnp.float32),
            pltpu.SMEM((1, 1), jnp.int32),
        ],
    )(x, afW1, afb1, afW2r, afb2, asW1b, asb1, row0, asW2r, asb2,
      at1W, at2_W, at2b, vW, vb, g1, g2, g3)

    return (actor, critic.reshape(1, N, VF))


# speedup vs baseline: 1.1757x; 1.1757x over previous
import jax
import jax.numpy as jnp
from jax import lax
from jax.experimental import pallas as pl
from jax.experimental.pallas import tpu as pltpu

F = 768
N = 8192
VF = 64
TILE = 512
NT = N // TILE
ACT = 2 * N + 2


def _body(x_ref, afW1_ref, afb1_ref, afW2r_ref, afb2_ref,
          asW1b_ref, asb1_ref, row0_ref, asW2r_ref, asb2_ref,
          at1W_ref, at2W_ref, at2b_ref, vW_ref, vb_ref,
          g1_ref, g2_ref, g3_ref,
          critic_ref, actor_ref,
          base2_scr, lg1_scr, lg2_scr, pooled_scr, af_scr):
    i = pl.program_id(0)

    @pl.when(i < NT)
    def _phase_a():
        x = x_ref[...]
        h1 = jnp.maximum(
            jnp.dot(x, afW1_ref[...], preferred_element_type=jnp.float32)
            + afb1_ref[...], 0.0)
        lg1_scr[:, pl.ds(i * TILE, TILE)] = lax.dot_general(
            afW2r_ref[...], h1,
            (((1,), (1,)), ((), ())), preferred_element_type=jnp.float32)

        base2_scr[i] = jnp.dot(
            x, asW1b_ref[...], preferred_element_type=jnp.float32) \
            + asb1_ref[...]

        ht = jnp.maximum(
            jnp.dot(x, at1W_ref[...], preferred_element_type=jnp.float32),
            0.0)
        psum = jnp.sum(ht, axis=0, keepdims=True)

        @pl.when(i == 0)
        def _():
            pooled_scr[...] = jnp.zeros_like(pooled_scr)

        pooled_scr[...] += psum

        critic_ref[...] = jnp.maximum(
            jnp.dot(x, vW_ref[...], preferred_element_type=jnp.float32)
            + vb_ref[...], 0.0)

        @pl.when(i == NT - 1)
        def _():
            z = lg1_scr[...] + afb2_ref[0, 0] + g1_ref[...]
            m = jnp.max(z)
            idx = lax.broadcasted_iota(jnp.int32, (1, N), 1)
            af_scr[0, 0] = jnp.min(jnp.where(z == m, idx, N))

    @pl.when(i >= NT)
    def _phase_b():
        j = i - NT
        a_f = af_scr[0, 0]
        c = a_f.astype(jnp.float32)
        h2 = jnp.maximum(base2_scr[j] + c * row0_ref[...], 0.0)
        lg2_scr[:, pl.ds(j * TILE, TILE)] = lax.dot_general(
            asW2r_ref[...], h2,
            (((1,), (1,)), ((), ())), preferred_element_type=jnp.float32)

        @pl.when(i == 2 * NT - 1)
        def _():
            idx = lax.broadcasted_iota(jnp.int32, (1, N), 1)
            z2 = jnp.where(idx == a_f, -jnp.inf,
                           lg2_scr[...] + asb2_ref[0, 0] + g2_ref[...])
            m2 = jnp.max(z2)
            a_s = jnp.min(jnp.where(z2 == m2, idx, N))

            lgt = jnp.dot(pooled_scr[...], at2W_ref[...],
                          preferred_element_type=jnp.float32) + at2b_ref[...]
            zt = lgt + g3_ref[...]
            a_t = jnp.where(zt[0, 1] > zt[0, 0], 1, 0)

            f1 = jnp.where(a_f == 0, 1.0, 0.0).astype(jnp.float32)
            f2 = jnp.where(a_s == 0, 1.0, 0.0).astype(jnp.float32)
            f3 = jnp.where(a_t == 0, 1.0, 0.0).astype(jnp.float32)
            aidx = lax.broadcasted_iota(jnp.int32, (1, ACT), 1)
            actor_ref[...] = jnp.where(
                aidx < N, f1, jnp.where(aidx < 2 * N, f2, f3))


def _full(shape):
    return pl.BlockSpec(shape, lambda i: tuple(0 for _ in shape))


def _smem11():
    return pl.BlockSpec((1, 1), lambda i: (0, 0), memory_space=pltpu.SMEM)


def kernel(features, af_W1, af_b1, af_W2, af_b2, as_W1, as_b1, as_W2,
           as_b2, at1_W, at2_W, at2_b, v_W, v_b):
    key = jax.random.key(42)
    k1, k2, k3 = jax.random.split(key, 3)
    g1 = jax.random.gumbel(k1, (1, N), jnp.float32)
    g2 = jax.random.gumbel(k2, (1, N), jnp.float32)
    g3 = jax.random.gumbel(k3, (1, 2), jnp.float32)

    x = features.reshape(N, F)
    afW1 = af_W1
    afb1 = af_b1.reshape(1, F)
    afW2r = af_W2.reshape(1, F)
    afb2 = af_b2.reshape(1, 1)
    asW1b = as_W1[1:]
    row0 = as_W1[0].reshape(1, F)
    asb1 = as_b1.reshape(1, F)
    asW2r = as_W2.reshape(1, F)
    asb2 = as_b2.reshape(1, 1)
    at1W = at1_W
    at2b = at2_b.reshape(1, 2)
    vW = v_W
    vb = v_b.reshape(1, VF)

    critic, actor = pl.pallas_call(
        _body,
        grid=(2 * NT,),
        in_specs=[
            pl.BlockSpec((TILE, F), lambda i: (jnp.minimum(i, NT - 1), 0)),
            _full((F, F)), _full((1, F)), _full((1, F)), _smem11(),
            _full((F, F)), _full((1, F)), _full((1, F)), _full((1, F)),
            _smem11(),
            _full((F, F)), _full((F, 2)), _full((1, 2)),
            _full((F, VF)), _full((1, VF)),
            _full((1, N)), _full((1, N)), _full((1, 2)),
        ],
        out_specs=[
            pl.BlockSpec((TILE, VF), lambda i: (jnp.minimum(i, NT - 1), 0)),
            _full((1, ACT)),
        ],
        out_shape=[
            jax.ShapeDtypeStruct((N, VF), jnp.float32),
            jax.ShapeDtypeStruct((1, ACT), jnp.float32),
        ],
        scratch_shapes=[
            pltpu.VMEM((NT, TILE, F), jnp.float32),
            pltpu.VMEM((1, N), jnp.float32),
            pltpu.VMEM((1, N), jnp.float32),
            pltpu.VMEM((1, F), jnp.float32),
            pltpu.SMEM((1, 1), jnp.int32),
        ],
    )(x, afW1, afb1, afW2r, afb2, asW1b, asb1, row0, asW2r, asb2,
      at1W, at2_W, at2b, vW, vb, g1, g2, g3)

    return (actor, critic.reshape(1, N, VF))


# speedup vs baseline: 1.2292x; 1.0455x over previous
import jax
import jax.numpy as jnp
from jax import lax
from jax.experimental import pallas as pl
from jax.experimental.pallas import tpu as pltpu

F = 768
N = 8192
VF = 64
TILE = 1024
NT = N // TILE
ACT = 2 * N + 2


def _body(x_ref, afW1_ref, afb1_ref, afW2r_ref, afb2_ref,
          asW1b_ref, asb1_ref, row0_ref, asW2r_ref, asb2_ref,
          at1W_ref, at2W_ref, at2b_ref, vW_ref, vb_ref,
          g1_ref, g2_ref, g3_ref,
          critic_ref, actor_ref,
          base2_scr, lg1_scr, lg2_scr, pooled_scr, af_scr):
    i = pl.program_id(0)

    @pl.when(i < NT)
    def _phase_a():
        x = x_ref[...]
        h1 = jnp.maximum(
            jnp.dot(x, afW1_ref[...], preferred_element_type=jnp.float32)
            + afb1_ref[...], 0.0)
        lg1_scr[:, pl.ds(i * TILE, TILE)] = lax.dot_general(
            afW2r_ref[...], h1,
            (((1,), (1,)), ((), ())), preferred_element_type=jnp.float32)

        base2_scr[i] = jnp.dot(
            x, asW1b_ref[...], preferred_element_type=jnp.float32) \
            + asb1_ref[...]

        ht = jnp.maximum(
            jnp.dot(x, at1W_ref[...], preferred_element_type=jnp.float32),
            0.0)
        psum = jnp.sum(ht, axis=0, keepdims=True)

        @pl.when(i == 0)
        def _():
            pooled_scr[...] = jnp.zeros_like(pooled_scr)

        pooled_scr[...] += psum

        critic_ref[...] = jnp.maximum(
            jnp.dot(x, vW_ref[...], preferred_element_type=jnp.float32)
            + vb_ref[...], 0.0)

        @pl.when(i == NT - 1)
        def _():
            z = lg1_scr[...] + afb2_ref[0, 0] + g1_ref[...]
            m = jnp.max(z)
            idx = lax.broadcasted_iota(jnp.int32, (1, N), 1)
            af_scr[0, 0] = jnp.min(jnp.where(z == m, idx, N))

    @pl.when(i >= NT)
    def _phase_b():
        j = i - NT
        a_f = af_scr[0, 0]
        c = a_f.astype(jnp.float32)
        h2 = jnp.maximum(base2_scr[j] + c * row0_ref[...], 0.0)
        lg2_scr[:, pl.ds(j * TILE, TILE)] = lax.dot_general(
            asW2r_ref[...], h2,
            (((1,), (1,)), ((), ())), preferred_element_type=jnp.float32)

        @pl.when(i == 2 * NT - 1)
        def _():
            idx = lax.broadcasted_iota(jnp.int32, (1, N), 1)
            z2 = jnp.where(idx == a_f, -jnp.inf,
                           lg2_scr[...] + asb2_ref[0, 0] + g2_ref[...])
            m2 = jnp.max(z2)
            a_s = jnp.min(jnp.where(z2 == m2, idx, N))

            lgt = jnp.dot(pooled_scr[...], at2W_ref[...],
                          preferred_element_type=jnp.float32) + at2b_ref[...]
            zt = lgt + g3_ref[...]
            a_t = jnp.where(zt[0, 1] > zt[0, 0], 1, 0)

            f1 = jnp.where(a_f == 0, 1.0, 0.0).astype(jnp.float32)
            f2 = jnp.where(a_s == 0, 1.0, 0.0).astype(jnp.float32)
            f3 = jnp.where(a_t == 0, 1.0, 0.0).astype(jnp.float32)
            aidx = lax.broadcasted_iota(jnp.int32, (1, ACT), 1)
            actor_ref[...] = jnp.where(
                aidx < N, f1, jnp.where(aidx < 2 * N, f2, f3))


def _full(shape):
    return pl.BlockSpec(shape, lambda i: tuple(0 for _ in shape))


def _smem11():
    return pl.BlockSpec((1, 1), lambda i: (0, 0), memory_space=pltpu.SMEM)


def kernel(features, af_W1, af_b1, af_W2, af_b2, as_W1, as_b1, as_W2,
           as_b2, at1_W, at2_W, at2_b, v_W, v_b):
    key = jax.random.key(42)
    k1, k2, k3 = jax.random.split(key, 3)
    g1 = jax.random.gumbel(k1, (1, N), jnp.float32)
    g2 = jax.random.gumbel(k2, (1, N), jnp.float32)
    g3 = jax.random.gumbel(k3, (1, 2), jnp.float32)

    x = features.reshape(N, F)
    afW1 = af_W1
    afb1 = af_b1.reshape(1, F)
    afW2r = af_W2.reshape(1, F)
    afb2 = af_b2.reshape(1, 1)
    asW1b = as_W1[1:]
    row0 = as_W1[0].reshape(1, F)
    asb1 = as_b1.reshape(1, F)
    asW2r = as_W2.reshape(1, F)
    asb2 = as_b2.reshape(1, 1)
    at1W = at1_W
    at2b = at2_b.reshape(1, 2)
    vW = v_W
    vb = v_b.reshape(1, VF)

    critic, actor = pl.pallas_call(
        _body,
        grid=(2 * NT,),
        in_specs=[
            pl.BlockSpec((TILE, F), lambda i: (jnp.minimum(i, NT - 1), 0)),
            _full((F, F)), _full((1, F)), _full((1, F)), _smem11(),
            _full((F, F)), _full((1, F)), _full((1, F)), _full((1, F)),
            _smem11(),
            _full((F, F)), _full((F, 2)), _full((1, 2)),
            _full((F, VF)), _full((1, VF)),
            _full((1, N)), _full((1, N)), _full((1, 2)),
        ],
        out_specs=[
            pl.BlockSpec((TILE, VF), lambda i: (jnp.minimum(i, NT - 1), 0)),
            _full((1, ACT)),
        ],
        out_shape=[
            jax.ShapeDtypeStruct((N, VF), jnp.float32),
            jax.ShapeDtypeStruct((1, ACT), jnp.float32),
        ],
        scratch_shapes=[
            pltpu.VMEM((NT, TILE, F), jnp.float32),
            pltpu.VMEM((1, N), jnp.float32),
            pltpu.VMEM((1, N), jnp.float32),
            pltpu.VMEM((1, F), jnp.float32),
            pltpu.SMEM((1, 1), jnp.int32),
        ],
    )(x, afW1, afb1, afW2r, afb2, asW1b, asb1, row0, asW2r, asb2,
      at1W, at2_W, at2b, vW, vb, g1, g2, g3)

    return (actor, critic.reshape(1, N, VF))


# speedup vs baseline: 1.4249x; 1.1592x over previous
import jax
import jax.numpy as jnp
from jax import lax
from jax.experimental import pallas as pl
from jax.experimental.pallas import tpu as pltpu

F = 768
N = 8192
VF = 64
TILE = 1024
NT = N // TILE
ACT = 2 * N + 2


def _body(x_ref, Wcat_ref, bcat_ref, afW2r_ref, afb2_ref,
          row0_ref, asW2r_ref, asb2_ref,
          at2W_ref, at2b_ref,
          g1_ref, g2_ref, g3_ref,
          critic_ref, actor_ref,
          base2_scr, lg1_scr, lg2_scr, pooled_scr, af_scr):
    i = pl.program_id(0)

    @pl.when(i < NT)
    def _phase_a():
        x = x_ref[...]
        y = jnp.dot(x, Wcat_ref[...], preferred_element_type=jnp.float32) \
            + bcat_ref[...]
        h1 = jnp.maximum(y[:, :F], 0.0)
        lg1_scr[:, pl.ds(i * TILE, TILE)] = lax.dot_general(
            afW2r_ref[...], h1,
            (((1,), (1,)), ((), ())), preferred_element_type=jnp.float32)

        base2_scr[i] = y[:, F:2 * F]

        ht = jnp.maximum(y[:, 2 * F:3 * F], 0.0)
        psum = jnp.sum(ht, axis=0, keepdims=True)

        @pl.when(i == 0)
        def _():
            pooled_scr[...] = jnp.zeros_like(pooled_scr)

        pooled_scr[...] += psum

        critic_ref[...] = jnp.maximum(y[:, 3 * F:3 * F + VF], 0.0)

        @pl.when(i == NT - 1)
        def _():
            z = lg1_scr[...] + afb2_ref[0, 0] + g1_ref[...]
            m = jnp.max(z)
            idx = lax.broadcasted_iota(jnp.int32, (1, N), 1)
            af_scr[0, 0] = jnp.min(jnp.where(z == m, idx, N))

    @pl.when(i >= NT)
    def _phase_b():
        j = i - NT
        a_f = af_scr[0, 0]
        c = a_f.astype(jnp.float32)
        h2 = jnp.maximum(base2_scr[j] + c * row0_ref[...], 0.0)
        lg2_scr[:, pl.ds(j * TILE, TILE)] = lax.dot_general(
            asW2r_ref[...], h2,
            (((1,), (1,)), ((), ())), preferred_element_type=jnp.float32)

        @pl.when(i == 2 * NT - 1)
        def _():
            idx = lax.broadcasted_iota(jnp.int32, (1, N), 1)
            z2 = jnp.where(idx == a_f, -jnp.inf,
                           lg2_scr[...] + asb2_ref[0, 0] + g2_ref[...])
            m2 = jnp.max(z2)
            a_s = jnp.min(jnp.where(z2 == m2, idx, N))

            lgt = jnp.dot(pooled_scr[...], at2W_ref[...],
                          preferred_element_type=jnp.float32) + at2b_ref[...]
            zt = lgt + g3_ref[...]
            a_t = jnp.where(zt[0, 1] > zt[0, 0], 1, 0)

            f1 = jnp.where(a_f == 0, 1.0, 0.0).astype(jnp.float32)
            f2 = jnp.where(a_s == 0, 1.0, 0.0).astype(jnp.float32)
            f3 = jnp.where(a_t == 0, 1.0, 0.0).astype(jnp.float32)
            aidx = lax.broadcasted_iota(jnp.int32, (1, ACT), 1)
            actor_ref[...] = jnp.where(
                aidx < N, f1, jnp.where(aidx < 2 * N, f2, f3))


def _full(shape):
    return pl.BlockSpec(shape, lambda i: tuple(0 for _ in shape))


def _smem11():
    return pl.BlockSpec((1, 1), lambda i: (0, 0), memory_space=pltpu.SMEM)


def kernel(features, af_W1, af_b1, af_W2, af_b2, as_W1, as_b1, as_W2,
           as_b2, at1_W, at2_W, at2_b, v_W, v_b):
    key = jax.random.key(42)
    k1, k2, k3 = jax.random.split(key, 3)
    g1 = jax.random.gumbel(k1, (1, N), jnp.float32)
    g2 = jax.random.gumbel(k2, (1, N), jnp.float32)
    g3 = jax.random.gumbel(k3, (1, 2), jnp.float32)

    x = features.reshape(N, F)
    Wcat = jnp.concatenate([af_W1, as_W1[1:], at1_W, v_W], axis=1)
    bcat = jnp.concatenate(
        [af_b1, as_b1, jnp.zeros((F,), jnp.float32), v_b]).reshape(1, -1)
    afW2r = af_W2.reshape(1, F)
    afb2 = af_b2.reshape(1, 1)
    row0 = as_W1[0].reshape(1, F)
    asW2r = as_W2.reshape(1, F)
    asb2 = as_b2.reshape(1, 1)
    at2b = at2_b.reshape(1, 2)
    vb = v_b.reshape(1, VF)

    critic, actor = pl.pallas_call(
        _body,
        grid=(2 * NT,),
        in_specs=[
            pl.BlockSpec((TILE, F), lambda i: (jnp.minimum(i, NT - 1), 0)),
            _full((F, 3 * F + VF)), _full((1, 3 * F + VF)),
            _full((1, F)), _smem11(),
            _full((1, F)), _full((1, F)), _smem11(),
            _full((F, 2)), _full((1, 2)),
            _full((1, N)), _full((1, N)), _full((1, 2)),
        ],
        out_specs=[
            pl.BlockSpec((TILE, VF), lambda i: (jnp.minimum(i, NT - 1), 0)),
            _full((1, ACT)),
        ],
        out_shape=[
            jax.ShapeDtypeStruct((N, VF), jnp.float32),
            jax.ShapeDtypeStruct((1, ACT), jnp.float32),
        ],
        scratch_shapes=[
            pltpu.VMEM((NT, TILE, F), jnp.float32),
            pltpu.VMEM((1, N), jnp.float32),
            pltpu.VMEM((1, N), jnp.float32),
            pltpu.VMEM((1, F), jnp.float32),
            pltpu.SMEM((1, 1), jnp.int32),
        ],
    )(x, Wcat, bcat, afW2r, afb2, row0, asW2r, asb2,
      at2_W, at2b, g1, g2, g3)

    return (actor, critic.reshape(1, N, VF))


# speedup vs baseline: 1.4344x; 1.0066x over previous
import jax
import jax.numpy as jnp
from jax import lax
from jax.experimental import pallas as pl
from jax.experimental.pallas import tpu as pltpu

F = 768
N = 8192
VF = 64
TILE = 1024
NT = N // TILE
ACT = 2 * N + 2


def _body(x_ref, Wcat_ref, bcat_ref, afW2r_ref, afb2_ref,
          row0_ref, asW2r_ref, asb2_ref,
          at2W_ref, at2b_ref,
          g1_ref, g2_ref, g3_ref,
          critic_ref, actor_ref,
          base2_scr, lg1_scr, lg2_scr, pooled_scr, af_scr):
    i = pl.program_id(0)

    @pl.when(i < NT)
    def _phase_a():
        x = x_ref[...].astype(jnp.bfloat16)
        y = jnp.dot(x, Wcat_ref[...], preferred_element_type=jnp.float32) \
            + bcat_ref[...]
        h1 = jnp.maximum(y[:, :F], 0.0)
        lg1_scr[:, pl.ds(i * TILE, TILE)] = lax.dot_general(
            afW2r_ref[...], h1,
            (((1,), (1,)), ((), ())), preferred_element_type=jnp.float32)

        base2_scr[i] = y[:, F:2 * F]

        ht = jnp.maximum(y[:, 2 * F:3 * F], 0.0)
        psum = jnp.sum(ht, axis=0, keepdims=True)

        @pl.when(i == 0)
        def _():
            pooled_scr[...] = jnp.zeros_like(pooled_scr)

        pooled_scr[...] += psum

        critic_ref[...] = jnp.maximum(y[:, 3 * F:3 * F + VF], 0.0)

        @pl.when(i == NT - 1)
        def _():
            z = lg1_scr[...] + afb2_ref[0, 0] + g1_ref[...]
            m = jnp.max(z)
            idx = lax.broadcasted_iota(jnp.int32, (1, N), 1)
            af_scr[0, 0] = jnp.min(jnp.where(z == m, idx, N))

    @pl.when(i >= NT)
    def _phase_b():
        j = i - NT
        a_f = af_scr[0, 0]
        c = a_f.astype(jnp.float32)
        h2 = jnp.maximum(base2_scr[j] + c * row0_ref[...], 0.0)
        lg2_scr[:, pl.ds(j * TILE, TILE)] = lax.dot_general(
            asW2r_ref[...], h2,
            (((1,), (1,)), ((), ())), preferred_element_type=jnp.float32)

        @pl.when(i == 2 * NT - 1)
        def _():
            idx = lax.broadcasted_iota(jnp.int32, (1, N), 1)
            z2 = jnp.where(idx == a_f, -jnp.inf,
                           lg2_scr[...] + asb2_ref[0, 0] + g2_ref[...])
            m2 = jnp.max(z2)
            a_s = jnp.min(jnp.where(z2 == m2, idx, N))

            lgt = jnp.dot(pooled_scr[...], at2W_ref[...],
                          preferred_element_type=jnp.float32) + at2b_ref[...]
            zt = lgt + g3_ref[...]
            a_t = jnp.where(zt[0, 1] > zt[0, 0], 1, 0)

            f1 = jnp.where(a_f == 0, 1.0, 0.0).astype(jnp.float32)
            f2 = jnp.where(a_s == 0, 1.0, 0.0).astype(jnp.float32)
            f3 = jnp.where(a_t == 0, 1.0, 0.0).astype(jnp.float32)
            aidx = lax.broadcasted_iota(jnp.int32, (1, ACT), 1)
            actor_ref[...] = jnp.where(
                aidx < N, f1, jnp.where(aidx < 2 * N, f2, f3))


def _full(shape):
    return pl.BlockSpec(shape, lambda i: tuple(0 for _ in shape))


def _smem11():
    return pl.BlockSpec((1, 1), lambda i: (0, 0), memory_space=pltpu.SMEM)


def kernel(features, af_W1, af_b1, af_W2, af_b2, as_W1, as_b1, as_W2,
           as_b2, at1_W, at2_W, at2_b, v_W, v_b):
    key = jax.random.key(42)
    k1, k2, k3 = jax.random.split(key, 3)
    g1 = jax.random.gumbel(k1, (1, N), jnp.float32)
    g2 = jax.random.gumbel(k2, (1, N), jnp.float32)
    g3 = jax.random.gumbel(k3, (1, 2), jnp.float32)

    x = features.reshape(N, F)
    Wcat = jnp.concatenate(
        [af_W1, as_W1[1:], at1_W, v_W], axis=1).astype(jnp.bfloat16)
    bcat = jnp.concatenate(
        [af_b1, as_b1, jnp.zeros((F,), jnp.float32), v_b]).reshape(1, -1)
    afW2r = af_W2.reshape(1, F)
    afb2 = af_b2.reshape(1, 1)
    row0 = as_W1[0].reshape(1, F)
    asW2r = as_W2.reshape(1, F)
    asb2 = as_b2.reshape(1, 1)
    at2b = at2_b.reshape(1, 2)
    vb = v_b.reshape(1, VF)

    critic, actor = pl.pallas_call(
        _body,
        grid=(2 * NT,),
        in_specs=[
            pl.BlockSpec((TILE, F), lambda i: (jnp.minimum(i, NT - 1), 0)),
            pl.BlockSpec((F, 3 * F + VF), lambda i: (0, 0)),
            _full((1, 3 * F + VF)),
            _full((1, F)), _smem11(),
            _full((1, F)), _full((1, F)), _smem11(),
            _full((F, 2)), _full((1, 2)),
            _full((1, N)), _full((1, N)), _full((1, 2)),
        ],
        out_specs=[
            pl.BlockSpec((TILE, VF), lambda i: (jnp.minimum(i, NT - 1), 0)),
            _full((1, ACT)),
        ],
        out_shape=[
            jax.ShapeDtypeStruct((N, VF), jnp.float32),
            jax.ShapeDtypeStruct((1, ACT), jnp.float32),
        ],
        scratch_shapes=[
            pltpu.VMEM((NT, TILE, F), jnp.float32),
            pltpu.VMEM((1, N), jnp.float32),
            pltpu.VMEM((1, N), jnp.float32),
            pltpu.VMEM((1, F), jnp.float32),
            pltpu.SMEM((1, 1), jnp.int32),
        ],
    )(x, Wcat, bcat, afW2r, afb2, row0, asW2r, asb2,
      at2_W, at2b, g1, g2, g3)

    return (actor, critic.reshape(1, N, VF))


# speedup vs baseline: 1.4414x; 1.0049x over previous
import jax
import jax.numpy as jnp
from jax import lax
from jax.experimental import pallas as pl
from jax.experimental.pallas import tpu as pltpu

F = 768
N = 8192
VF = 64
TILE = 1024
NT = N // TILE
ACT = 2 * N + 2


def _body(x_ref, afW1_ref, asW1b_ref, at1W_ref, vW_ref,
          bcat_ref, afW2r_ref, afb2_ref,
          row0_ref, asW2r_ref, asb2_ref,
          at2W_ref, at2b_ref,
          g1_ref, g2_ref, g3_ref,
          critic_ref, actor_ref,
          Wcat_scr, base2_scr, lg1_scr, lg2_scr, pooled_scr, af_scr):
    i = pl.program_id(0)

    @pl.when(i == 0)
    def _stage():
        Wcat_scr[:, :F] = afW1_ref[...].astype(jnp.bfloat16)
        Wcat_scr[:, F:2 * F] = asW1b_ref[...].astype(jnp.bfloat16)
        Wcat_scr[:, 2 * F:3 * F] = at1W_ref[...].astype(jnp.bfloat16)
        Wcat_scr[:, 3 * F:] = vW_ref[...].astype(jnp.bfloat16)

    @pl.when(i < NT)
    def _phase_a():
        x = x_ref[...].astype(jnp.bfloat16)
        y = jnp.dot(x, Wcat_scr[...], preferred_element_type=jnp.float32) \
            + bcat_ref[...]
        h1 = jnp.maximum(y[:, :F], 0.0)
        lg1_scr[:, pl.ds(i * TILE, TILE)] = lax.dot_general(
            afW2r_ref[...], h1,
            (((1,), (1,)), ((), ())), preferred_element_type=jnp.float32)

        base2_scr[i] = y[:, F:2 * F]

        ht = jnp.maximum(y[:, 2 * F:3 * F], 0.0)
        psum = jnp.sum(ht, axis=0, keepdims=True)

        @pl.when(i == 0)
        def _():
            pooled_scr[...] = jnp.zeros_like(pooled_scr)

        pooled_scr[...] += psum

        critic_ref[...] = jnp.maximum(y[:, 3 * F:3 * F + VF], 0.0)

        @pl.when(i == NT - 1)
        def _():
            z = lg1_scr[...] + afb2_ref[0, 0] + g1_ref[...]
            m = jnp.max(z)
            idx = lax.broadcasted_iota(jnp.int32, (1, N), 1)
            af_scr[0, 0] = jnp.min(jnp.where(z == m, idx, N))

    @pl.when(i >= NT)
    def _phase_b():
        j = i - NT
        a_f = af_scr[0, 0]
        c = a_f.astype(jnp.float32)
        h2 = jnp.maximum(base2_scr[j] + c * row0_ref[...], 0.0)
        lg2_scr[:, pl.ds(j * TILE, TILE)] = lax.dot_general(
            asW2r_ref[...], h2,
            (((1,), (1,)), ((), ())), preferred_element_type=jnp.float32)

        @pl.when(i == 2 * NT - 1)
        def _():
            idx = lax.broadcasted_iota(jnp.int32, (1, N), 1)
            z2 = jnp.where(idx == a_f, -jnp.inf,
                           lg2_scr[...] + asb2_ref[0, 0] + g2_ref[...])
            m2 = jnp.max(z2)
            a_s = jnp.min(jnp.where(z2 == m2, idx, N))

            lgt = jnp.dot(pooled_scr[...], at2W_ref[...],
                          preferred_element_type=jnp.float32) + at2b_ref[...]
            zt = lgt + g3_ref[...]
            a_t = jnp.where(zt[0, 1] > zt[0, 0], 1, 0)

            f1 = jnp.where(a_f == 0, 1.0, 0.0).astype(jnp.float32)
            f2 = jnp.where(a_s == 0, 1.0, 0.0).astype(jnp.float32)
            f3 = jnp.where(a_t == 0, 1.0, 0.0).astype(jnp.float32)
            aidx = lax.broadcasted_iota(jnp.int32, (1, ACT), 1)
            actor_ref[...] = jnp.where(
                aidx < N, f1, jnp.where(aidx < 2 * N, f2, f3))


def _full(shape):
    return pl.BlockSpec(shape, lambda i: tuple(0 for _ in shape))


def _smem11():
    return pl.BlockSpec((1, 1), lambda i: (0, 0), memory_space=pltpu.SMEM)


def kernel(features, af_W1, af_b1, af_W2, af_b2, as_W1, as_b1, as_W2,
           as_b2, at1_W, at2_W, at2_b, v_W, v_b):
    key = jax.random.key(42)
    k1, k2, k3 = jax.random.split(key, 3)
    g1 = jax.random.gumbel(k1, (1, N), jnp.float32)
    g2 = jax.random.gumbel(k2, (1, N), jnp.float32)
    g3 = jax.random.gumbel(k3, (1, 2), jnp.float32)

    x = features.reshape(N, F)
    bcat = jnp.concatenate(
        [af_b1, as_b1, jnp.zeros((F,), jnp.float32), v_b]).reshape(1, -1)
    afW2r = af_W2.reshape(1, F)
    afb2 = af_b2.reshape(1, 1)
    row0 = as_W1[0].reshape(1, F)
    asW2r = as_W2.reshape(1, F)
    asb2 = as_b2.reshape(1, 1)
    at2b = at2_b.reshape(1, 2)
    vb = v_b.reshape(1, VF)

    critic, actor = pl.pallas_call(
        _body,
        grid=(2 * NT,),
        in_specs=[
            pl.BlockSpec((TILE, F), lambda i: (jnp.minimum(i, NT - 1), 0)),
            _full((F, F)), _full((F, F)), _full((F, F)), _full((F, VF)),
            _full((1, 3 * F + VF)),
            _full((1, F)), _smem11(),
            _full((1, F)), _full((1, F)), _smem11(),
            _full((F, 2)), _full((1, 2)),
            _full((1, N)), _full((1, N)), _full((1, 2)),
        ],
        out_specs=[
            pl.BlockSpec((TILE, VF), lambda i: (jnp.minimum(i, NT - 1), 0)),
            _full((1, ACT)),
        ],
        out_shape=[
            jax.ShapeDtypeStruct((N, VF), jnp.float32),
            jax.ShapeDtypeStruct((1, ACT), jnp.float32),
        ],
        scratch_shapes=[
            pltpu.VMEM((F, 3 * F + VF), jnp.bfloat16),
            pltpu.VMEM((NT, TILE, F), jnp.float32),
            pltpu.VMEM((1, N), jnp.float32),
            pltpu.VMEM((1, N), jnp.float32),
            pltpu.VMEM((1, F), jnp.float32),
            pltpu.SMEM((1, 1), jnp.int32),
        ],
    )(x, af_W1, as_W1[1:], at1_W, v_W, bcat, afW2r, afb2, row0, asW2r,
      asb2, at2_W, at2b, g1, g2, g3)

    return (actor, critic.reshape(1, N, VF))
